# baseline (speedup 1.0000x reference)
import functools

import jax
import jax.numpy as jnp
from jax import lax
from jax.experimental import pallas as pl
from jax.experimental.pallas import tpu as pltpu
from jax.experimental.pallas import tpu_sc as plsc

_D = 64
_B = 4096 * 200
_NW = 32
_BLK = 128
_NBLK = _B // _BLK // _NW
_BPW = _NBLK * _BLK
_HB = 4096 // _BLK


def _body(table_hbm, x_hbm, out_hbm, idx_v, rows_v, tbuf, gs0, gs1, os0, os1):
    wid = lax.axis_index("s") * 2 + lax.axis_index("c")
    base = wid * _BPW
    pltpu.sync_copy(x_hbm.at[pl.ds(base, _BPW)], idx_v)

    gsems = (gs0, gs1)
    osems = (os0, os1)
    iota16 = lax.iota(jnp.int32, 16)
    dh_idx = [(iota16 + 16 * k) >> 3 for k in range(4)]
    dl_idx = [(iota16 + 16 * k) & 7 for k in range(4)]

    def sg(g, slot):
        off = pl.multiple_of(g * _BLK, _BLK)
        pltpu.async_copy(
            table_hbm.at[idx_v.at[pl.ds(off, _BLK)]],
            rows_v.at[slot],
            gsems[slot],
        )

    def wg(slot):
        pltpu.make_async_copy(
            table_hbm.at[pl.ds(0, _BLK)], rows_v.at[slot], gsems[slot]
        ).wait()

    def tr(slot):
        @plsc.parallel_loop(0, _BLK, unroll=16)
        def _(b):
            colb = jnp.full((16,), b, jnp.int32)
            for k in range(4):
                vals = rows_v[slot, b, pl.ds(16 * k, 16)]
                plsc.store_scatter(
                    tbuf.at[slot], [dh_idx[k], dl_idx[k], colb], vals
                )

    def so(g, slot):
        t = wid * _NBLK + g
        h = t // _HB
        bh = t % _HB
        pltpu.async_copy(
            tbuf.at[slot, slice(None), slice(None), pl.ds(0, 128)],
            out_hbm.at[h, slice(None), bh],
            osems[slot],
        )

    def wo(slot):
        pltpu.make_async_copy(
            tbuf.at[slot, slice(None), slice(None), pl.ds(0, 128)],
            out_hbm.at[0, slice(None), 0],
            osems[slot],
        ).wait()

    sg(0, 0)
    sg(1, 1)
    wg(0)
    tr(0)
    so(0, 0)
    sg(2, 0)
    wg(1)
    tr(1)
    so(1, 1)
    sg(3, 1)

    @pl.loop(2, _NBLK - 2, step=2)
    def _(g):
        wg(0)
        wo(0)
        tr(0)
        so(g, 0)
        sg(g + 2, 0)
        wg(1)
        wo(1)
        tr(1)
        so(g + 1, 1)
        sg(g + 3, 1)

    wg(0)
    wo(0)
    tr(0)
    so(_NBLK - 2, 0)
    wg(1)
    wo(1)
    tr(1)
    so(_NBLK - 1, 1)
    wo(0)
    wo(1)


_embed_gather = functools.partial(
    pl.kernel,
    out_type=jax.ShapeDtypeStruct((200, 8, _HB, 8, 128), jnp.float32),
    mesh=plsc.VectorSubcoreMesh(core_axis_name="c", subcore_axis_name="s"),
    scratch_types=[
        pltpu.VMEM((_BPW,), jnp.int32),
        pltpu.VMEM((2, _BLK, _D), jnp.float32),
        pltpu.VMEM((2, 8, 8, 133), jnp.float32),
        pltpu.SemaphoreType.DMA,
        pltpu.SemaphoreType.DMA,
        pltpu.SemaphoreType.DMA,
        pltpu.SemaphoreType.DMA,
    ],
    compiler_params=pltpu.CompilerParams(use_tc_tiling_on_sc=False, needs_layout_passes=False),
)(_body)


def kernel(x, table):
    xt = x.T.reshape(-1)
    o5 = _embed_gather(table, xt)
    return jnp.transpose(o5, (2, 4, 0, 1, 3)).reshape(
        x.shape + (table.shape[1],)
    )


# speedup vs baseline: 1.0107x; 1.0107x over previous
import functools

import jax
import jax.numpy as jnp
from jax import lax
from jax.experimental import pallas as pl
from jax.experimental.pallas import tpu as pltpu
from jax.experimental.pallas import tpu_sc as plsc

_D = 64
_B = 4096 * 200
_NW = 32
_BLK = 128
_NBLK = _B // _BLK // _NW
_BPW = _NBLK * _BLK
_HB = 4096 // _BLK


def _body(table_hbm, x_hbm, out_hbm, idx_v, rows_v, tbuf, gs0, gs1, os0, os1):
    wid = lax.axis_index("s") * 2 + lax.axis_index("c")
    base = wid * _BPW
    pltpu.sync_copy(x_hbm.at[pl.ds(base, _BPW)], idx_v)

    gsems = (gs0, gs1)
    osems = (os0, os1)
    iota16 = lax.iota(jnp.int32, 16)
    dh_idx = [(iota16 + 16 * k) >> 3 for k in range(4)]
    dl_idx = [(iota16 + 16 * k) & 7 for k in range(4)]

    def sg(g, slot):
        off = pl.multiple_of(g * _BLK, _BLK)
        pltpu.async_copy(
            table_hbm.at[idx_v.at[pl.ds(off, _BLK)]],
            rows_v.at[slot],
            gsems[slot],
        )

    def wg(slot):
        pltpu.make_async_copy(
            table_hbm.at[pl.ds(0, _BLK)], rows_v.at[slot], gsems[slot]
        ).wait()

    def tr(slot):
        @plsc.parallel_loop(0, _BLK, unroll=8)
        def _(b):
            colb = jnp.full((16,), b, jnp.int32)
            for k in range(4):
                vals = rows_v[slot, b, pl.ds(16 * k, 16)]
                plsc.store_scatter(
                    tbuf.at[slot], [dh_idx[k], dl_idx[k], colb], vals
                )

    def so(g, slot):
        t = wid * _NBLK + g
        h = t // _HB
        bh = t % _HB
        pltpu.async_copy(
            tbuf.at[slot, slice(None), slice(None), pl.ds(0, 128)],
            out_hbm.at[h, slice(None), bh],
            osems[slot],
        )

    def wo(slot):
        pltpu.make_async_copy(
            tbuf.at[slot, slice(None), slice(None), pl.ds(0, 128)],
            out_hbm.at[0, slice(None), 0],
            osems[slot],
        ).wait()

    sg(0, 0)
    sg(1, 1)
    wg(0)
    tr(0)
    so(0, 0)
    sg(2, 0)
    wg(1)
    tr(1)
    so(1, 1)
    sg(3, 1)

    @pl.loop(2, _NBLK - 2, step=2)
    def _(g):
        wg(0)
        wo(0)
        tr(0)
        so(g, 0)
        sg(g + 2, 0)
        wg(1)
        wo(1)
        tr(1)
        so(g + 1, 1)
        sg(g + 3, 1)

    wg(0)
    wo(0)
    tr(0)
    so(_NBLK - 2, 0)
    wg(1)
    wo(1)
    tr(1)
    so(_NBLK - 1, 1)
    wo(0)
    wo(1)


_embed_gather = functools.partial(
    pl.kernel,
    out_type=jax.ShapeDtypeStruct((200, 8, _HB, 8, 128), jnp.float32),
    mesh=plsc.VectorSubcoreMesh(core_axis_name="c", subcore_axis_name="s"),
    scratch_types=[
        pltpu.VMEM((_BPW,), jnp.int32),
        pltpu.VMEM((2, _BLK, _D), jnp.float32),
        pltpu.VMEM((2, 8, 8, 133), jnp.float32),
        pltpu.SemaphoreType.DMA,
        pltpu.SemaphoreType.DMA,
        pltpu.SemaphoreType.DMA,
        pltpu.SemaphoreType.DMA,
    ],
    compiler_params=pltpu.CompilerParams(use_tc_tiling_on_sc=False, needs_layout_passes=False),
)(_body)


def kernel(x, table):
    xt = x.T.reshape(-1)
    o5 = _embed_gather(table, xt)
    return jnp.transpose(o5, (2, 4, 0, 1, 3)).reshape(
        x.shape + (table.shape[1],)
    )
